# baseline (speedup 1.0000x reference)
import functools

import jax
import jax.numpy as jnp
from jax import lax
from jax.experimental import pallas as pl
from jax.experimental.pallas import tpu as pltpu
from jax.experimental.pallas import tpu_sc as plsc

N = 10000
NPAD = 10240
E = 320000
EPAD = 327680
P = 128
CT = 8
MAX_LOGSTD = 10.0

NC = 2
NS = 16
NWK = NC * NS
ER = EPAD // 128
ER_W = ER // NWK
LR = 2 * ER
LR_W = LR // NWK
STRIPE = NPAD // NS
BLK = 512
NBLK = NPAD // BLK

_f32 = jnp.float32
_i32 = jnp.int32


def _sc_degree_body(c2, w2, out, cidx_v, w_v, deg_v):
    cid = lax.axis_index("c")
    sid = lax.axis_index("s")
    wid = cid * NS + sid

    def zero(i, _):
        deg_v[pl.ds(i * 16, 16)] = jnp.zeros((16,), _f32)
        return 0

    lax.fori_loop(0, NPAD // 16, zero, 0, unroll=16)

    base = wid * ER_W

    def chunk(t, _):
        rb = base + t * 16
        pltpu.sync_copy(c2.at[pl.ds(rb, 16)], cidx_v)
        pltpu.sync_copy(w2.at[pl.ds(rb, 16)], w_v)

        def inner(k, _):
            i = k // 8
            j = k - i * 8
            cv = cidx_v[i, pl.ds(j * 16, 16)]
            wv = w_v[i, pl.ds(j * 16, 16)]
            plsc.addupdate_scatter(deg_v, [cv], wv)
            return 0

        lax.fori_loop(0, 128, inner, 0, unroll=8)
        return 0

    lax.fori_loop(0, ER_W // 16, chunk, 0)
    pltpu.sync_copy(deg_v, out.at[wid])


AGG_CR = 8
AGG_NCH = ER_W // AGG_CR


def _sc_aggregate_body(src, r2, c2, w2, dis, out,
                       ridx_v, cidx_v, ew_v, norm_v, rows_v, rows2_v, dis_v,
                       src_sh, acc, gsem, ssem):
    cid = lax.axis_index("c")
    sid = lax.axis_index("s")
    wid = cid * NS + sid

    zeros16 = jnp.zeros((16,), _f32)

    def zero(i, _):
        rows_v[i, pl.ds(0, 16)] = zeros16
        rows_v[i, pl.ds(16, 16)] = zeros16
        return 0

    lax.fori_loop(0, STRIPE, zero, 0, unroll=8)
    pltpu.sync_copy(rows_v.at[pl.ds(0, STRIPE)],
                    acc.at[pl.ds(sid * STRIPE, STRIPE)])
    pltpu.sync_copy(src.at[pl.ds(sid * STRIPE, STRIPE)],
                    src_sh.at[pl.ds(sid * STRIPE, STRIPE)])
    pltpu.sync_copy(dis, dis_v)
    plsc.subcore_barrier()

    base = wid * ER_W

    def load_idx(t):
        h = (t % 2) * AGG_CR
        rb = base + t * AGG_CR
        pltpu.sync_copy(r2.at[pl.ds(rb, AGG_CR)],
                        ridx_v.at[pl.ds(h, AGG_CR)])
        pltpu.sync_copy(c2.at[pl.ds(rb, AGG_CR)],
                        cidx_v.at[pl.ds(h, AGG_CR)])
        pltpu.sync_copy(w2.at[pl.ds(rb, AGG_CR)],
                        ew_v.at[pl.ds(h, AGG_CR)])

    def fire_gathers(t):
        h = (t % 2) * AGG_CR
        return [pltpu.async_copy(src_sh.at[ridx_v.at[h + i]],
                                 rows_v.at[pl.ds(i * 128, 128)], gsem)
                for i in range(AGG_CR)]

    def fire_scatters(t):
        h = (t % 2) * AGG_CR
        return [pltpu.async_copy(rows2_v.at[pl.ds(i * 128, 128)],
                                 acc.at[cidx_v.at[h + i]], ssem, add=True)
                for i in range(AGG_CR)]

    def norm(t):
        h = (t % 2) * AGG_CR

        def nrm(k, _):
            i = k // 8
            j = k - i * 8
            rv = ridx_v[h + i, pl.ds(j * 16, 16)]
            cv = cidx_v[h + i, pl.ds(j * 16, 16)]
            wv = ew_v[h + i, pl.ds(j * 16, 16)]
            dr = plsc.load_gather(dis_v, [rv])
            dc = plsc.load_gather(dis_v, [cv])
            norm_v[pl.ds(k * 16, 16)] = dr * wv * dc
            return 0

        lax.fori_loop(0, AGG_CR * 8, nrm, 0, unroll=4)

    def scale():
        def sc(e, _):
            m = plsc.load_gather(norm_v, [jnp.full((16,), e, _i32)])
            rows2_v[e, pl.ds(0, 16)] = rows_v[e, pl.ds(0, 16)] * m
            rows2_v[e, pl.ds(16, 16)] = rows_v[e, pl.ds(16, 16)] * m
            return 0

        lax.fori_loop(0, AGG_CR * 128, sc, 0, unroll=8)

    load_idx(0)
    gath = fire_gathers(0)
    scat = []
    for t in range(AGG_NCH):
        norm(t)
        for d in gath:
            d.wait()
        for d in scat:
            d.wait()
        scale()
        scat = fire_scatters(t)
        if t + 1 < AGG_NCH:
            load_idx(t + 1)
            gath = fire_gathers(t + 1)
    for d in scat:
        d.wait()
    plsc.subcore_barrier()
    pltpu.sync_copy(acc.at[pl.ds(sid * STRIPE, STRIPE)],
                    out.at[cid, pl.ds(sid * STRIPE, STRIPE)])


LS_CR = 4
LS_NCH = LR_W // LS_CR


def _sc_edge_loss_body(g, lr2, lc2, lw2, lm2, out,
                       ridx_v, cidx_v, w_v, m_v, gr_v, gc_v, dots_v, accv,
                       g_sh, sem0, sem1):
    cid = lax.axis_index("c")
    sid = lax.axis_index("s")
    wid = cid * NS + sid
    base = wid * LR_W

    pltpu.sync_copy(g.at[pl.ds(sid * STRIPE, STRIPE)],
                    g_sh.at[pl.ds(sid * STRIPE, STRIPE)])
    plsc.subcore_barrier()

    def load_idx(t, h):
        rb = base + t * LS_CR
        pltpu.sync_copy(lr2.at[pl.ds(rb, LS_CR)],
                        ridx_v.at[pl.ds(h, LS_CR)])
        pltpu.sync_copy(lc2.at[pl.ds(rb, LS_CR)],
                        cidx_v.at[pl.ds(h, LS_CR)])
        pltpu.sync_copy(lw2.at[pl.ds(rb, LS_CR)],
                        w_v.at[pl.ds(h, LS_CR)])
        pltpu.sync_copy(lm2.at[pl.ds(rb, LS_CR)],
                        m_v.at[pl.ds(h, LS_CR)])

    def gather_descs(h, eh, sem):
        cps = []
        for i in range(LS_CR):
            cps.append(pltpu.make_async_copy(
                g_sh.at[ridx_v.at[h + i]],
                gr_v.at[pl.ds(eh + i * 128, 128)], sem))
            cps.append(pltpu.make_async_copy(
                g_sh.at[cidx_v.at[h + i]],
                gc_v.at[pl.ds(eh + i * 128, 128)], sem))
        return cps

    lane = lax.broadcasted_iota(_i32, (16,), 0)
    lane15 = lane == 15
    perms = [lane ^ 8, lane ^ 4, lane ^ 2, lane ^ 1]

    def dots(eh):
        def dot(e, _):
            ee = eh + e
            r00, r01 = plsc.unpack(gr_v[ee, pl.ds(0, 32)],
                                   format=plsc.PackFormat.INTERLEAVED)
            r10, r11 = plsc.unpack(gr_v[ee, pl.ds(32, 32)],
                                   format=plsc.PackFormat.INTERLEAVED)
            c00, c01 = plsc.unpack(gc_v[ee, pl.ds(0, 32)],
                                   format=plsc.PackFormat.INTERLEAVED)
            c10, c11 = plsc.unpack(gc_v[ee, pl.ds(32, 32)],
                                   format=plsc.PackFormat.INTERLEAVED)
            v = (r00 * c00 + r01 * c01) + (r10 * c10 + r11 * c11)
            for pm in perms:
                v = v + v.at[pm].get(mode="promise_in_bounds")
            plsc.store_scatter(dots_v, [jnp.full((16,), e, _i32)], v,
                               mask=lane15)
            return 0

        lax.fori_loop(0, LS_CR * 128, dot, 0, unroll=8)

    def sig(h, acc):
        def body(k, a):
            i = k // 8
            j = k - i * 8
            d = dots_v[pl.ds(k * 16, 16)]
            wv = w_v[h + i, pl.ds(j * 16, 16)]
            mv = m_v[h + i, pl.ds(j * 16, 16)]
            s = 1.0 / (1.0 + jnp.exp(-d))
            tt = s - wv
            return a + mv * tt * tt

        return lax.fori_loop(0, LS_CR * 8, body, acc, unroll=8)

    load_idx(0, 0)
    for d in gather_descs(0, 0, sem0):
        d.start()

    def chunk(t, par, acc):
        h = par * LS_CR
        eh = par * LS_CR * 128
        hn = (1 - par) * LS_CR
        ehn = (1 - par) * LS_CR * 128

        @pl.when(t + 1 < LS_NCH)
        def _():
            load_idx(t + 1, hn)
            for d in gather_descs(hn, ehn, sem1 if par == 0 else sem0):
                d.start()

        for d in gather_descs(h, eh, sem0 if par == 0 else sem1):
            d.wait()
        dots(eh)
        return sig(h, acc)

    def two_chunks(u, acc):
        acc = chunk(2 * u, 0, acc)
        return chunk(2 * u + 1, 1, acc)

    acc = lax.fori_loop(0, LS_NCH // 2, two_chunks, jnp.zeros((16,), _f32))
    accv[...] = acc
    pltpu.sync_copy(accv, out.at[wid])


@functools.cache
def _get_sc_kernels():
    mesh = plsc.VectorSubcoreMesh(
        core_axis_name="c", subcore_axis_name="s",
        num_cores=NC, num_subcores=NS)
    cparams = pltpu.CompilerParams(
        needs_layout_passes=False, use_tc_tiling_on_sc=False)
    sc_degree = pl.kernel(
        _sc_degree_body,
        out_type=jax.ShapeDtypeStruct((NWK, NPAD), _f32),
        mesh=mesh,
        scratch_types=[
            pltpu.VMEM((16, 128), _i32),
            pltpu.VMEM((16, 128), _f32),
            pltpu.VMEM((NPAD,), _f32),
        ],
        compiler_params=cparams,
    )
    sc_aggregate = pl.kernel(
        _sc_aggregate_body,
        out_type=jax.ShapeDtypeStruct((NC, NPAD, 32), _f32),
        mesh=mesh,
        scratch_types=[
            pltpu.VMEM((16, 128), _i32),
            pltpu.VMEM((16, 128), _i32),
            pltpu.VMEM((16, 128), _f32),
            pltpu.VMEM((AGG_CR * 128,), _f32),
            pltpu.VMEM((AGG_CR * 128, 32), _f32),
            pltpu.VMEM((AGG_CR * 128, 32), _f32),
            pltpu.VMEM((NPAD,), _f32),
            pltpu.VMEM_SHARED((NPAD, 32), _f32),
            pltpu.VMEM_SHARED((NPAD, 32), _f32),
            pltpu.SemaphoreType.DMA,
            pltpu.SemaphoreType.DMA,
        ],
        compiler_params=cparams,
    )
    sc_edge_loss = pl.kernel(
        _sc_edge_loss_body,
        out_type=jax.ShapeDtypeStruct((NWK, 16), _f32),
        mesh=mesh,
        scratch_types=[
            pltpu.VMEM((8, 128), _i32),
            pltpu.VMEM((8, 128), _i32),
            pltpu.VMEM((8, 128), _f32),
            pltpu.VMEM((8, 128), _f32),
            pltpu.VMEM((1024, 64), jnp.bfloat16),
            pltpu.VMEM((1024, 64), jnp.bfloat16),
            pltpu.VMEM((512,), _f32),
            pltpu.VMEM((16,), _f32),
            pltpu.VMEM_SHARED((NPAD, 64), jnp.bfloat16),
            pltpu.SemaphoreType.DMA,
            pltpu.SemaphoreType.DMA,
        ],
        compiler_params=cparams,
    )
    return sc_degree, sc_aggregate, sc_edge_loss


def _tc_encoder_body(x_ref, w1_ref, b1_ref, w2_ref, b2_ref, o_ref):
    h = x_ref[...] @ w1_ref[...] + b1_ref[...]
    h = jnp.where(h > 0, h, jnp.exp(h) - 1.0)
    f = h @ w2_ref[...] + b2_ref[...]
    f = jnp.where(f > 0, f, jnp.exp(f) - 1.0)
    o_ref[...] = jnp.pad(f, ((0, 0), (0, 12)))


def _tc_encoder(xp, w1, b1, w2, b2):
    return pl.pallas_call(
        _tc_encoder_body,
        grid=(NBLK,),
        in_specs=[
            pl.BlockSpec((BLK, P), lambda i: (i, 0)),
            pl.BlockSpec((P, 100), lambda i: (0, 0)),
            pl.BlockSpec((1, 100), lambda i: (0, 0)),
            pl.BlockSpec((100, 20), lambda i: (0, 0)),
            pl.BlockSpec((1, 20), lambda i: (0, 0)),
        ],
        out_specs=pl.BlockSpec((BLK, 32), lambda i: (i, 0)),
        out_shape=jax.ShapeDtypeStruct((NPAD, 32), _f32),
    )(xp, w1, b1, w2, b2)


def _tc_prep_body(dp_ref, dis_ref, disq_ref):
    deg = jnp.sum(dp_ref[...], axis=0) + 1.0
    dis = lax.rsqrt(deg)
    dis_ref[...] = dis
    disq_ref[...] = dis * dis


def _tc_prep(deg_parts):
    return pl.pallas_call(
        _tc_prep_body,
        grid=(1,),
        in_specs=[pl.BlockSpec((NWK, 80, 128), lambda i: (0, 0, 0))],
        out_specs=[
            pl.BlockSpec((80, 128), lambda i: (0, 0)),
            pl.BlockSpec((80, 128), lambda i: (0, 0)),
        ],
        out_shape=[
            jax.ShapeDtypeStruct((80, 128), _f32),
            jax.ShapeDtypeStruct((80, 128), _f32),
        ],
    )(deg_parts.reshape(NWK, 80, 128))


def _tc_conv1_body(ap_ref, fx_ref, dq_ref, w_ref, b_ref, o_ref):
    agg = ap_ref[0] + ap_ref[1] + dq_ref[...] * fx_ref[...]
    h = agg @ w_ref[...] + b_ref[...]
    o_ref[...] = jnp.maximum(h, 0.0)


def _tc_conv1(a1p, fxp, disq_col, w32, b32):
    return pl.pallas_call(
        _tc_conv1_body,
        grid=(NBLK,),
        in_specs=[
            pl.BlockSpec((NC, BLK, 32), lambda i: (0, i, 0)),
            pl.BlockSpec((BLK, 32), lambda i: (i, 0)),
            pl.BlockSpec((BLK, 1), lambda i: (i, 0)),
            pl.BlockSpec((32, 32), lambda i: (0, 0)),
            pl.BlockSpec((1, 32), lambda i: (0, 0)),
        ],
        out_specs=pl.BlockSpec((BLK, 32), lambda i: (i, 0)),
        out_shape=jax.ShapeDtypeStruct((NPAD, 32), _f32),
    )(a1p, fxp, disq_col, w32, b32)


def _tc_main_body(fx_ref, h_ref, ap_ref, dq_ref,
                  wm_ref, bm_ref, wl_ref, bl_ref,
                  wc_ref, bc_ref, wd_ref, bd_ref,
                  wx_ref, bx_ref, wg_ref, bg_ref,
                  ct_ref, xd_ref, pw_ref, fg_ref, kl_ref):
    agg2 = ap_ref[0] + ap_ref[1] + dq_ref[...] * h_ref[...]
    mu = agg2 @ wm_ref[...] + bm_ref[...]
    ls = jnp.minimum(agg2 @ wl_ref[...] + bl_ref[...], MAX_LOGSTD)
    feat = jnp.concatenate([fx_ref[...][:, :20], mu], axis=1)

    ct = jnp.maximum(feat @ wc_ref[...] + bc_ref[...], 0.0)
    ct_ref[...] = ct / (jnp.sum(ct, axis=1, keepdims=True) + 1e-6)

    pw2 = jnp.maximum(feat @ wd_ref[...] + bd_ref[...], 0.0)
    s = (pw2[:, 0:128] + pw2[:, 128:256] + pw2[:, 256:384] + pw2[:, 384:512]
         + pw2[:, 512:640] + pw2[:, 640:768] + pw2[:, 768:896]
         + pw2[:, 896:1024])
    inv = 1.0 / (s + 1e-6)
    pwn = pw2 * jnp.concatenate([inv] * 8, axis=1)
    pw_ref[...] = jnp.transpose(pwn.reshape(BLK, 8, 128),
                                (0, 2, 1)).reshape(BLK, 1024)

    xd_ref[...] = feat @ wx_ref[...] + bx_ref[...]
    fg_ref[...] = jnp.maximum(feat @ wg_ref[...] + bg_ref[...], 0.0).astype(
        jnp.bfloat16)

    ex = jnp.exp(ls)
    klt = jnp.sum(1.0 + 2.0 * ls - mu * mu - ex * ex, axis=1, keepdims=True)
    rowid = pl.program_id(0) * BLK + lax.broadcasted_iota(_i32, (BLK, 1), 0)
    kl_ref[...] = jnp.where(rowid < N, klt, 0.0)


def _tc_main(fxp, hp, a2p, disq_col, wm, bm, wl, bl, wc, bc, wd2, bd2,
             wx, bx, wg, bg):
    return pl.pallas_call(
        _tc_main_body,
        grid=(NBLK,),
        in_specs=[
            pl.BlockSpec((BLK, 32), lambda i: (i, 0)),
            pl.BlockSpec((BLK, 32), lambda i: (i, 0)),
            pl.BlockSpec((NC, BLK, 32), lambda i: (0, i, 0)),
            pl.BlockSpec((BLK, 1), lambda i: (i, 0)),
            pl.BlockSpec((32, 20), lambda i: (0, 0)),
            pl.BlockSpec((1, 20), lambda i: (0, 0)),
            pl.BlockSpec((32, 20), lambda i: (0, 0)),
            pl.BlockSpec((1, 20), lambda i: (0, 0)),
            pl.BlockSpec((40, CT), lambda i: (0, 0)),
            pl.BlockSpec((1, CT), lambda i: (0, 0)),
            pl.BlockSpec((40, 1024), lambda i: (0, 0)),
            pl.BlockSpec((1, 1024), lambda i: (0, 0)),
            pl.BlockSpec((40, P), lambda i: (0, 0)),
            pl.BlockSpec((1, P), lambda i: (0, 0)),
            pl.BlockSpec((40, 64), lambda i: (0, 0)),
            pl.BlockSpec((1, 64), lambda i: (0, 0)),
        ],
        out_specs=[
            pl.BlockSpec((BLK, CT), lambda i: (i, 0)),
            pl.BlockSpec((BLK, P), lambda i: (i, 0)),
            pl.BlockSpec((BLK, 1024), lambda i: (i, 0)),
            pl.BlockSpec((BLK, 64), lambda i: (i, 0)),
            pl.BlockSpec((BLK, 1), lambda i: (i, 0)),
        ],
        out_shape=[
            jax.ShapeDtypeStruct((NPAD, CT), _f32),
            jax.ShapeDtypeStruct((NPAD, P), _f32),
            jax.ShapeDtypeStruct((NPAD, 1024), _f32),
            jax.ShapeDtypeStruct((NPAD, 64), jnp.bfloat16),
            jax.ShapeDtypeStruct((NPAD, 1), _f32),
        ],
    )(fxp, hp, a2p, disq_col, wm, bm, wl, bl, wc, bc, wd2, bd2, wx, bx, wg, bg)


def kernel(x, edge_index, edge_weight, enc_W1, enc_b1, enc_W2, enc_b2,
           gcf_W, gcf_b, gcm_W, gcm_b, gcl_W, gcl_b, dec_W, dec_b,
           fg_W, fg_b, cls_W, cls_b, dcv_W, dcv_b):
    row = edge_index[0]
    col = edge_index[1]

    pad_e = EPAD - E
    r2 = jnp.pad(row, (0, pad_e)).reshape(ER, 128)
    c2 = jnp.pad(col, (0, pad_e)).reshape(ER, 128)
    w2 = jnp.pad(edge_weight, (0, pad_e)).reshape(ER, 128)

    neg = jax.random.randint(jax.random.key(123), (2, E), 0, N)
    mask1 = jnp.pad(jnp.ones((E,), _f32), (0, pad_e))
    lr2 = jnp.concatenate([r2, jnp.pad(neg[0], (0, pad_e)).reshape(ER, 128)])
    lc2 = jnp.concatenate([c2, jnp.pad(neg[1], (0, pad_e)).reshape(ER, 128)])
    lw2 = jnp.concatenate([w2, jnp.zeros((ER, 128), _f32)])
    lm2 = jnp.concatenate([mask1.reshape(ER, 128), mask1.reshape(ER, 128)])

    xp = jnp.pad(x, ((0, NPAD - N), (0, 0)))
    b1 = enc_b1.reshape(1, 100)
    b2 = enc_b2.reshape(1, 20)

    gcf_W32 = jnp.pad(gcf_W, ((0, 12), (0, 0)))
    gcm_W32 = jnp.pad(gcm_W, ((0, 0), (0, 0)))
    gcl_W32 = jnp.pad(gcl_W, ((0, 0), (0, 0)))

    dcv_W2 = dcv_W.reshape(40, 128, 8).transpose(0, 2, 1).reshape(40, 1024)
    dcv_b2 = dcv_b.reshape(128, 8).transpose(1, 0).reshape(1, 1024)

    sc_degree, sc_aggregate, sc_edge_loss = _get_sc_kernels()
    feat_xp = _tc_encoder(xp, enc_W1, b1, enc_W2, b2)
    deg_parts = sc_degree(c2, w2)
    dis80, disq80 = _tc_prep(deg_parts)
    dis = dis80.reshape(NPAD)
    disq_col = disq80.reshape(NPAD, 1)

    a1p = sc_aggregate(feat_xp, r2, c2, w2, dis)
    hp = _tc_conv1(a1p, feat_xp, disq_col, gcf_W32, gcf_b.reshape(1, 32))

    a2p = sc_aggregate(hp, r2, c2, w2, dis)
    ctp, xd, pw2, fg, kl = _tc_main(
        feat_xp, hp, a2p, disq_col,
        gcm_W32, gcm_b.reshape(1, 20), gcl_W32, gcl_b.reshape(1, 20),
        cls_W, cls_b.reshape(1, CT), dcv_W2, dcv_b2,
        dec_W, dec_b.reshape(1, P), fg_W, fg_b.reshape(1, 64))

    loss_parts = sc_edge_loss(fg, lr2, lc2, lw2, lm2)

    ct_perc = ctp[:N]
    x_dec = xd[:N]
    protein_weight = pw2[:N]
    kl_sum = jnp.sum(kl)
    edge_sum = jnp.sum(loss_parts)
    gae_loss = edge_sum / E + (-0.5 * kl_sum / N) / N
    return (ct_perc, x_dec, gae_loss, protein_weight)


# speedup vs baseline: 1.1665x; 1.1665x over previous
import functools

import jax
import jax.numpy as jnp
from jax import lax
from jax.experimental import pallas as pl
from jax.experimental.pallas import tpu as pltpu
from jax.experimental.pallas import tpu_sc as plsc

N = 10000
NPAD = 10240
E = 320000
EPAD = 327680
P = 128
CT = 8
MAX_LOGSTD = 10.0

NC = 2
NS = 16
NWK = NC * NS
ER = EPAD // 128
ER_W = ER // NWK
LR = 2 * ER
LR_W = LR // NWK
STRIPE = NPAD // NS
BLK = 512
NBLK = NPAD // BLK

_f32 = jnp.float32
_i32 = jnp.int32


def _sc_degree_body(c2, w2, out, cidx_v, w_v, deg_v):
    cid = lax.axis_index("c")
    sid = lax.axis_index("s")
    wid = cid * NS + sid

    def zero(i, _):
        deg_v[pl.ds(i * 16, 16)] = jnp.zeros((16,), _f32)
        return 0

    lax.fori_loop(0, NPAD // 16, zero, 0, unroll=16)

    base = wid * ER_W

    def chunk(t, _):
        rb = base + t * 16
        pltpu.sync_copy(c2.at[pl.ds(rb, 16)], cidx_v)
        pltpu.sync_copy(w2.at[pl.ds(rb, 16)], w_v)

        def inner(k, _):
            i = k // 8
            j = k - i * 8
            cv = cidx_v[i, pl.ds(j * 16, 16)]
            wv = w_v[i, pl.ds(j * 16, 16)]
            plsc.addupdate_scatter(deg_v, [cv], wv)
            return 0

        lax.fori_loop(0, 128, inner, 0, unroll=8)
        return 0

    lax.fori_loop(0, ER_W // 16, chunk, 0)
    pltpu.sync_copy(deg_v, out.at[wid])


AGG_CR = 8
AGG_NCH = ER_W // AGG_CR


def _sc_aggregate_body(src, r2, c2, w2, dis, out,
                       ridx_v, cidx_v, ew_v, norm_v, rows_v, rows2_v, dis_v,
                       src_sh, acc, gsem, ssem):
    cid = lax.axis_index("c")
    sid = lax.axis_index("s")
    wid = cid * NS + sid

    zeros16 = jnp.zeros((16,), _f32)

    def zero(i, _):
        rows_v[i, pl.ds(0, 16)] = zeros16
        rows_v[i, pl.ds(16, 16)] = zeros16
        return 0

    lax.fori_loop(0, STRIPE, zero, 0, unroll=8)
    pltpu.sync_copy(rows_v.at[pl.ds(0, STRIPE)],
                    acc.at[pl.ds(sid * STRIPE, STRIPE)])
    pltpu.sync_copy(src.at[pl.ds(sid * STRIPE, STRIPE)],
                    src_sh.at[pl.ds(sid * STRIPE, STRIPE)])
    pltpu.sync_copy(dis, dis_v)
    plsc.subcore_barrier()

    base = wid * ER_W

    def load_idx(t):
        h = (t % 2) * AGG_CR
        rb = base + t * AGG_CR
        pltpu.sync_copy(r2.at[pl.ds(rb, AGG_CR)],
                        ridx_v.at[pl.ds(h, AGG_CR)])
        pltpu.sync_copy(c2.at[pl.ds(rb, AGG_CR)],
                        cidx_v.at[pl.ds(h, AGG_CR)])
        pltpu.sync_copy(w2.at[pl.ds(rb, AGG_CR)],
                        ew_v.at[pl.ds(h, AGG_CR)])

    def fire_gathers(t):
        h = (t % 2) * AGG_CR
        return [pltpu.async_copy(src_sh.at[ridx_v.at[h + i]],
                                 rows_v.at[pl.ds(i * 128, 128)], gsem)
                for i in range(AGG_CR)]

    def fire_scatters(t):
        h = (t % 2) * AGG_CR
        return [pltpu.async_copy(rows2_v.at[pl.ds(i * 128, 128)],
                                 acc.at[cidx_v.at[h + i]], ssem, add=True)
                for i in range(AGG_CR)]

    def norm(t):
        h = (t % 2) * AGG_CR

        def nrm(k, _):
            i = k // 8
            j = k - i * 8
            rv = ridx_v[h + i, pl.ds(j * 16, 16)]
            cv = cidx_v[h + i, pl.ds(j * 16, 16)]
            wv = ew_v[h + i, pl.ds(j * 16, 16)]
            dr = plsc.load_gather(dis_v, [rv])
            dc = plsc.load_gather(dis_v, [cv])
            norm_v[pl.ds(k * 16, 16)] = dr * wv * dc
            return 0

        lax.fori_loop(0, AGG_CR * 8, nrm, 0, unroll=4)

    def scale():
        def sc(e, _):
            m = plsc.load_gather(norm_v, [jnp.full((16,), e, _i32)])
            rows2_v[e, pl.ds(0, 16)] = rows_v[e, pl.ds(0, 16)] * m
            rows2_v[e, pl.ds(16, 16)] = rows_v[e, pl.ds(16, 16)] * m
            return 0

        lax.fori_loop(0, AGG_CR * 128, sc, 0, unroll=8)

    load_idx(0)
    gath = fire_gathers(0)
    scat = []
    for t in range(AGG_NCH):
        norm(t)
        for d in gath:
            d.wait()
        for d in scat:
            d.wait()
        scale()
        scat = fire_scatters(t)
        if t + 1 < AGG_NCH:
            load_idx(t + 1)
            gath = fire_gathers(t + 1)
    for d in scat:
        d.wait()
    plsc.subcore_barrier()
    pltpu.sync_copy(acc.at[pl.ds(sid * STRIPE, STRIPE)],
                    out.at[cid, pl.ds(sid * STRIPE, STRIPE)])


LS_CR = 4
LS_NCH = LR_W // LS_CR


def _sc_edge_loss_body(g, lr2, lc2, lw2, lm2, out,
                       ridx_v, cidx_v, w_v, m_v, gr_v, gc_v, dots_v, accv,
                       g_sh, sem0, sem1):
    cid = lax.axis_index("c")
    sid = lax.axis_index("s")
    wid = cid * NS + sid
    base = wid * LR_W

    pltpu.sync_copy(g.at[pl.ds(sid * STRIPE, STRIPE)],
                    g_sh.at[pl.ds(sid * STRIPE, STRIPE)])
    plsc.subcore_barrier()

    def load_idx(t, h):
        rb = base + t * LS_CR
        pltpu.sync_copy(lr2.at[pl.ds(rb, LS_CR)],
                        ridx_v.at[pl.ds(h, LS_CR)])
        pltpu.sync_copy(lc2.at[pl.ds(rb, LS_CR)],
                        cidx_v.at[pl.ds(h, LS_CR)])
        pltpu.sync_copy(lw2.at[pl.ds(rb, LS_CR)],
                        w_v.at[pl.ds(h, LS_CR)])
        pltpu.sync_copy(lm2.at[pl.ds(rb, LS_CR)],
                        m_v.at[pl.ds(h, LS_CR)])

    def gather_descs(h, eh, sem):
        cps = []
        for i in range(LS_CR):
            cps.append(pltpu.make_async_copy(
                g_sh.at[ridx_v.at[h + i]],
                gr_v.at[pl.ds(eh + i * 128, 128)], sem))
            cps.append(pltpu.make_async_copy(
                g_sh.at[cidx_v.at[h + i]],
                gc_v.at[pl.ds(eh + i * 128, 128)], sem))
        return cps

    lane = lax.broadcasted_iota(_i32, (16,), 0)
    lane15 = lane == 15
    perms = [lane ^ 8, lane ^ 4, lane ^ 2, lane ^ 1]

    def dots(eh):
        def dot(e, _):
            ee = eh + e
            r00, r01 = plsc.unpack(gr_v[ee, pl.ds(0, 32)],
                                   format=plsc.PackFormat.INTERLEAVED)
            r10, r11 = plsc.unpack(gr_v[ee, pl.ds(32, 32)],
                                   format=plsc.PackFormat.INTERLEAVED)
            c00, c01 = plsc.unpack(gc_v[ee, pl.ds(0, 32)],
                                   format=plsc.PackFormat.INTERLEAVED)
            c10, c11 = plsc.unpack(gc_v[ee, pl.ds(32, 32)],
                                   format=plsc.PackFormat.INTERLEAVED)
            v = (r00 * c00 + r01 * c01) + (r10 * c10 + r11 * c11)
            for pm in perms:
                v = v + v.at[pm].get(mode="promise_in_bounds")
            plsc.store_scatter(dots_v, [jnp.full((16,), e, _i32)], v,
                               mask=lane15)
            return 0

        lax.fori_loop(0, LS_CR * 128, dot, 0, unroll=8)

    def sig(h, acc):
        def body(k, a):
            i = k // 8
            j = k - i * 8
            d = dots_v[pl.ds(k * 16, 16)]
            wv = w_v[h + i, pl.ds(j * 16, 16)]
            mv = m_v[h + i, pl.ds(j * 16, 16)]
            s = 1.0 / (1.0 + jnp.exp(-d))
            tt = s - wv
            return a + mv * tt * tt

        return lax.fori_loop(0, LS_CR * 8, body, acc, unroll=8)

    load_idx(0, 0)
    for d in gather_descs(0, 0, sem0):
        d.start()

    def chunk(t, par, acc):
        h = par * LS_CR
        eh = par * LS_CR * 128
        hn = (1 - par) * LS_CR
        ehn = (1 - par) * LS_CR * 128

        @pl.when(t + 1 < LS_NCH)
        def _():
            load_idx(t + 1, hn)
            for d in gather_descs(hn, ehn, sem1 if par == 0 else sem0):
                d.start()

        for d in gather_descs(h, eh, sem0 if par == 0 else sem1):
            d.wait()
        dots(eh)
        return sig(h, acc)

    def two_chunks(u, acc):
        acc = chunk(2 * u, 0, acc)
        return chunk(2 * u + 1, 1, acc)

    acc = lax.fori_loop(0, LS_NCH // 2, two_chunks, jnp.zeros((16,), _f32))
    accv[...] = acc
    pltpu.sync_copy(accv, out.at[wid])


@functools.cache
def _get_sc_kernels():
    mesh = plsc.VectorSubcoreMesh(
        core_axis_name="c", subcore_axis_name="s",
        num_cores=NC, num_subcores=NS)
    cparams = pltpu.CompilerParams(
        needs_layout_passes=False, use_tc_tiling_on_sc=False)
    sc_degree = pl.kernel(
        _sc_degree_body,
        out_type=jax.ShapeDtypeStruct((NWK, NPAD), _f32),
        mesh=mesh,
        scratch_types=[
            pltpu.VMEM((16, 128), _i32),
            pltpu.VMEM((16, 128), _f32),
            pltpu.VMEM((NPAD,), _f32),
        ],
        compiler_params=cparams,
    )
    sc_aggregate = pl.kernel(
        _sc_aggregate_body,
        out_type=jax.ShapeDtypeStruct((NC, NPAD, 32), _f32),
        mesh=mesh,
        scratch_types=[
            pltpu.VMEM((16, 128), _i32),
            pltpu.VMEM((16, 128), _i32),
            pltpu.VMEM((16, 128), _f32),
            pltpu.VMEM((AGG_CR * 128,), _f32),
            pltpu.VMEM((AGG_CR * 128, 32), _f32),
            pltpu.VMEM((AGG_CR * 128, 32), _f32),
            pltpu.VMEM((NPAD,), _f32),
            pltpu.VMEM_SHARED((NPAD, 32), _f32),
            pltpu.VMEM_SHARED((NPAD, 32), _f32),
            pltpu.SemaphoreType.DMA,
            pltpu.SemaphoreType.DMA,
        ],
        compiler_params=cparams,
    )
    sc_edge_loss = pl.kernel(
        _sc_edge_loss_body,
        out_type=jax.ShapeDtypeStruct((NWK, 16), _f32),
        mesh=mesh,
        scratch_types=[
            pltpu.VMEM((8, 128), _i32),
            pltpu.VMEM((8, 128), _i32),
            pltpu.VMEM((8, 128), _f32),
            pltpu.VMEM((8, 128), _f32),
            pltpu.VMEM((1024, 64), jnp.bfloat16),
            pltpu.VMEM((1024, 64), jnp.bfloat16),
            pltpu.VMEM((512,), _f32),
            pltpu.VMEM((16,), _f32),
            pltpu.VMEM_SHARED((NPAD, 64), jnp.bfloat16),
            pltpu.SemaphoreType.DMA,
            pltpu.SemaphoreType.DMA,
        ],
        compiler_params=cparams,
    )
    return sc_degree, sc_aggregate, sc_edge_loss


def _tc_encoder_body(x_ref, w1_ref, b1_ref, w2_ref, b2_ref, o_ref):
    h = x_ref[...] @ w1_ref[...] + b1_ref[...]
    h = jnp.where(h > 0, h, jnp.exp(h) - 1.0)
    f = h @ w2_ref[...] + b2_ref[...]
    f = jnp.where(f > 0, f, jnp.exp(f) - 1.0)
    o_ref[...] = jnp.pad(f, ((0, 0), (0, 12)))


def _tc_encoder(xp, w1, b1, w2, b2):
    return pl.pallas_call(
        _tc_encoder_body,
        grid=(NBLK,),
        in_specs=[
            pl.BlockSpec((BLK, P), lambda i: (i, 0)),
            pl.BlockSpec((P, 100), lambda i: (0, 0)),
            pl.BlockSpec((1, 100), lambda i: (0, 0)),
            pl.BlockSpec((100, 20), lambda i: (0, 0)),
            pl.BlockSpec((1, 20), lambda i: (0, 0)),
        ],
        out_specs=pl.BlockSpec((BLK, 32), lambda i: (i, 0)),
        out_shape=jax.ShapeDtypeStruct((NPAD, 32), _f32),
    )(xp, w1, b1, w2, b2)


def _tc_prep_body(dp_ref, dis_ref, disq_ref):
    deg = jnp.sum(dp_ref[...], axis=0) + 1.0
    dis = lax.rsqrt(deg)
    dis_ref[...] = dis
    disq_ref[...] = dis * dis


def _tc_prep(deg_parts):
    return pl.pallas_call(
        _tc_prep_body,
        grid=(1,),
        in_specs=[pl.BlockSpec((NWK, 80, 128), lambda i: (0, 0, 0))],
        out_specs=[
            pl.BlockSpec((80, 128), lambda i: (0, 0)),
            pl.BlockSpec((80, 128), lambda i: (0, 0)),
        ],
        out_shape=[
            jax.ShapeDtypeStruct((80, 128), _f32),
            jax.ShapeDtypeStruct((80, 128), _f32),
        ],
    )(deg_parts.reshape(NWK, 80, 128))


def _tc_conv1_body(ap_ref, fx_ref, dq_ref, w_ref, b_ref, o_ref):
    agg = ap_ref[0] + ap_ref[1] + dq_ref[...] * fx_ref[...]
    h = agg @ w_ref[...] + b_ref[...]
    o_ref[...] = jnp.maximum(h, 0.0)


def _tc_conv1(a1p, fxp, disq_col, w32, b32):
    return pl.pallas_call(
        _tc_conv1_body,
        grid=(NBLK,),
        in_specs=[
            pl.BlockSpec((NC, BLK, 32), lambda i: (0, i, 0)),
            pl.BlockSpec((BLK, 32), lambda i: (i, 0)),
            pl.BlockSpec((BLK, 1), lambda i: (i, 0)),
            pl.BlockSpec((32, 32), lambda i: (0, 0)),
            pl.BlockSpec((1, 32), lambda i: (0, 0)),
        ],
        out_specs=pl.BlockSpec((BLK, 32), lambda i: (i, 0)),
        out_shape=jax.ShapeDtypeStruct((NPAD, 32), _f32),
    )(a1p, fxp, disq_col, w32, b32)


def _tc_main_body(fx_ref, h_ref, ap_ref, dq_ref,
                  wm_ref, bm_ref, wl_ref, bl_ref,
                  wc_ref, bc_ref, wd_ref, bd_ref,
                  wx_ref, bx_ref, wg_ref, bg_ref,
                  ct_ref, xd_ref, pw_ref, fg_ref, kl_ref):
    agg2 = ap_ref[0] + ap_ref[1] + dq_ref[...] * h_ref[...]
    mu = agg2 @ wm_ref[...] + bm_ref[...]
    ls = jnp.minimum(agg2 @ wl_ref[...] + bl_ref[...], MAX_LOGSTD)
    feat = jnp.concatenate([fx_ref[...][:, :20], mu], axis=1)

    ct = jnp.maximum(feat @ wc_ref[...] + bc_ref[...], 0.0)
    ct_ref[...] = ct / (jnp.sum(ct, axis=1, keepdims=True) + 1e-6)

    pw2 = jnp.maximum(feat @ wd_ref[...] + bd_ref[...], 0.0)
    s = (pw2[:, 0:128] + pw2[:, 128:256] + pw2[:, 256:384] + pw2[:, 384:512]
         + pw2[:, 512:640] + pw2[:, 640:768] + pw2[:, 768:896]
         + pw2[:, 896:1024])
    inv = 1.0 / (s + 1e-6)
    pw_ref[...] = pw2 * jnp.concatenate([inv] * 8, axis=1)

    xd_ref[...] = feat @ wx_ref[...] + bx_ref[...]
    fg_ref[...] = jnp.maximum(feat @ wg_ref[...] + bg_ref[...], 0.0).astype(
        jnp.bfloat16)

    ex = jnp.exp(ls)
    klt = jnp.sum(1.0 + 2.0 * ls - mu * mu - ex * ex, axis=1, keepdims=True)
    rowid = pl.program_id(0) * BLK + lax.broadcasted_iota(_i32, (BLK, 1), 0)
    kl_ref[...] = jnp.where(rowid < N, klt, 0.0)


def _tc_main(fxp, hp, a2p, disq_col, wm, bm, wl, bl, wc, bc, wd2, bd2,
             wx, bx, wg, bg):
    return pl.pallas_call(
        _tc_main_body,
        grid=(NBLK,),
        in_specs=[
            pl.BlockSpec((BLK, 32), lambda i: (i, 0)),
            pl.BlockSpec((BLK, 32), lambda i: (i, 0)),
            pl.BlockSpec((NC, BLK, 32), lambda i: (0, i, 0)),
            pl.BlockSpec((BLK, 1), lambda i: (i, 0)),
            pl.BlockSpec((32, 20), lambda i: (0, 0)),
            pl.BlockSpec((1, 20), lambda i: (0, 0)),
            pl.BlockSpec((32, 20), lambda i: (0, 0)),
            pl.BlockSpec((1, 20), lambda i: (0, 0)),
            pl.BlockSpec((40, CT), lambda i: (0, 0)),
            pl.BlockSpec((1, CT), lambda i: (0, 0)),
            pl.BlockSpec((40, 1024), lambda i: (0, 0)),
            pl.BlockSpec((1, 1024), lambda i: (0, 0)),
            pl.BlockSpec((40, P), lambda i: (0, 0)),
            pl.BlockSpec((1, P), lambda i: (0, 0)),
            pl.BlockSpec((40, 64), lambda i: (0, 0)),
            pl.BlockSpec((1, 64), lambda i: (0, 0)),
        ],
        out_specs=[
            pl.BlockSpec((BLK, CT), lambda i: (i, 0)),
            pl.BlockSpec((BLK, P), lambda i: (i, 0)),
            pl.BlockSpec((BLK, 1024), lambda i: (i, 0)),
            pl.BlockSpec((BLK, 64), lambda i: (i, 0)),
            pl.BlockSpec((BLK, 1), lambda i: (i, 0)),
        ],
        out_shape=[
            jax.ShapeDtypeStruct((NPAD, CT), _f32),
            jax.ShapeDtypeStruct((NPAD, P), _f32),
            jax.ShapeDtypeStruct((NPAD, 1024), _f32),
            jax.ShapeDtypeStruct((NPAD, 64), jnp.bfloat16),
            jax.ShapeDtypeStruct((NPAD, 1), _f32),
        ],
    )(fxp, hp, a2p, disq_col, wm, bm, wl, bl, wc, bc, wd2, bd2, wx, bx, wg, bg)


def kernel(x, edge_index, edge_weight, enc_W1, enc_b1, enc_W2, enc_b2,
           gcf_W, gcf_b, gcm_W, gcm_b, gcl_W, gcl_b, dec_W, dec_b,
           fg_W, fg_b, cls_W, cls_b, dcv_W, dcv_b):
    row = edge_index[0]
    col = edge_index[1]

    pad_e = EPAD - E
    r2 = jnp.pad(row, (0, pad_e)).reshape(ER, 128)
    c2 = jnp.pad(col, (0, pad_e)).reshape(ER, 128)
    w2 = jnp.pad(edge_weight, (0, pad_e)).reshape(ER, 128)

    neg = jax.random.randint(jax.random.key(123), (2, E), 0, N)
    mask1 = jnp.pad(jnp.ones((E,), _f32), (0, pad_e))
    lr2 = jnp.concatenate([r2, jnp.pad(neg[0], (0, pad_e)).reshape(ER, 128)])
    lc2 = jnp.concatenate([c2, jnp.pad(neg[1], (0, pad_e)).reshape(ER, 128)])
    lw2 = jnp.concatenate([w2, jnp.zeros((ER, 128), _f32)])
    lm2 = jnp.concatenate([mask1.reshape(ER, 128), mask1.reshape(ER, 128)])

    xp = jnp.pad(x, ((0, NPAD - N), (0, 0)))
    b1 = enc_b1.reshape(1, 100)
    b2 = enc_b2.reshape(1, 20)

    gcf_W32 = jnp.pad(gcf_W, ((0, 12), (0, 0)))
    gcm_W32 = jnp.pad(gcm_W, ((0, 0), (0, 0)))
    gcl_W32 = jnp.pad(gcl_W, ((0, 0), (0, 0)))

    dcv_W2 = dcv_W.reshape(40, 128, 8).transpose(0, 2, 1).reshape(40, 1024)
    dcv_b2 = dcv_b.reshape(128, 8).transpose(1, 0).reshape(1, 1024)

    sc_degree, sc_aggregate, sc_edge_loss = _get_sc_kernels()
    feat_xp = _tc_encoder(xp, enc_W1, b1, enc_W2, b2)
    deg_parts = sc_degree(c2, w2)
    dis80, disq80 = _tc_prep(deg_parts)
    dis = dis80.reshape(NPAD)
    disq_col = disq80.reshape(NPAD, 1)

    a1p = sc_aggregate(feat_xp, r2, c2, w2, dis)
    hp = _tc_conv1(a1p, feat_xp, disq_col, gcf_W32, gcf_b.reshape(1, 32))

    a2p = sc_aggregate(hp, r2, c2, w2, dis)
    ctp, xd, pw2, fg, kl = _tc_main(
        feat_xp, hp, a2p, disq_col,
        gcm_W32, gcm_b.reshape(1, 20), gcl_W32, gcl_b.reshape(1, 20),
        cls_W, cls_b.reshape(1, CT), dcv_W2, dcv_b2,
        dec_W, dec_b.reshape(1, P), fg_W, fg_b.reshape(1, 64))

    loss_parts = sc_edge_loss(fg, lr2, lc2, lw2, lm2)

    ct_perc = ctp[:N]
    x_dec = xd[:N]
    protein_weight = (
        pw2[:N].reshape(N, 8, 128).transpose(0, 2, 1).reshape(N, P * CT))
    kl_sum = jnp.sum(kl)
    edge_sum = jnp.sum(loss_parts)
    gae_loss = edge_sum / E + (-0.5 * kl_sum / N) / N
    return (ct_perc, x_dec, gae_loss, protein_weight)
